# baseline (speedup 1.0000x reference)
import jax
import jax.numpy as jnp
from jax import lax
from jax.experimental import pallas as pl

EMBED_DIM = 384
D3 = EMBED_DIM // 3
N = 48
BB = 256

_INF = float("inf")


def _fold_kernel(wdist_ref, bdist_ref, emb_ref, wdens_ref, bdens_ref,
                 wout_ref, bout_ref, v_ref):
    wout = wout_ref[...]
    w_lo = wout[0:D3, :]
    w_mid = wout[D3:2 * D3, :]
    w_hi = wout[2 * D3:3 * D3, :]
    v1 = jnp.dot(wdist_ref[...], w_lo, preferred_element_type=jnp.float32)
    v2 = jnp.dot(wdens_ref[...], w_hi, preferred_element_type=jnp.float32)
    cvec = (jnp.dot(bdist_ref[...], w_lo, preferred_element_type=jnp.float32)
            + jnp.dot(emb_ref[...], w_mid, preferred_element_type=jnp.float32)
            + jnp.dot(bdens_ref[...], w_hi, preferred_element_type=jnp.float32)
            + bout_ref[...])
    zero = jnp.zeros((1, EMBED_DIM), jnp.float32)
    v_ref[...] = jnp.concatenate(
        [v1, v2, cvec, zero, zero, zero, zero, zero], axis=0)


def _block_kernel(pts_ref, v_ref, out_ref):
    x = pts_ref[0]
    y = pts_ref[1]
    z = pts_ref[2]

    cx = jnp.mean(x, axis=0, keepdims=True)
    cy = jnp.mean(y, axis=0, keepdims=True)
    cz = jnp.mean(z, axis=0, keepdims=True)
    cd = jnp.sqrt((x - cx) ** 2 + (y - cy) ** 2 + (z - cz) ** 2)

    m1 = jnp.full((N, BB), _INF, dtype=jnp.float32)
    m2 = m1
    m3 = m1
    row = lax.broadcasted_iota(jnp.int32, (N, BB), 0)
    for j in range(N):
        dx = x - x[j:j + 1, :]
        dy = y - y[j:j + 1, :]
        dz = z - z[j:j + 1, :]
        dsq = dx * dx + dy * dy + dz * dz
        dsq = jnp.where(row == j, _INF, dsq)
        a = jnp.maximum(m1, dsq)
        m1 = jnp.minimum(m1, dsq)
        b = jnp.maximum(m2, dsq)
        m2 = jnp.minimum(m2, a)
        m3 = jnp.minimum(m3, b)
    ld = (jnp.sqrt(m1) + jnp.sqrt(m2) + jnp.sqrt(m3)) * (1.0 / 3.0)

    v1 = v_ref[0:1, :]
    v2 = v_ref[1:2, :]
    cvec = v_ref[2:3, :]

    cd_t = jnp.transpose(cd, (1, 0))[:, :, None]
    ld_t = jnp.transpose(ld, (1, 0))[:, :, None]
    out_ref[...] = (cd_t * v1[None, :, :] + ld_t * v2[None, :, :]
                    + cvec[None, :, :])


def _build(interpret=False):
    def run(points, W_dist, b_dist, emb_count, W_dens, b_dens, W_out, b_out):
        Bv = points.shape[0]
        pts_t = jnp.transpose(points, (2, 1, 0))
        emb_row = emb_count[N:N + 1, :]
        vec_spec = pl.BlockSpec((1, D3), lambda: (0, 0))
        folded = pl.pallas_call(
            _fold_kernel,
            in_specs=[vec_spec, vec_spec, vec_spec, vec_spec, vec_spec,
                      pl.BlockSpec((EMBED_DIM, EMBED_DIM), lambda: (0, 0)),
                      pl.BlockSpec((1, EMBED_DIM), lambda: (0, 0))],
            out_specs=pl.BlockSpec((8, EMBED_DIM), lambda: (0, 0)),
            out_shape=jax.ShapeDtypeStruct((8, EMBED_DIM), jnp.float32),
            interpret=interpret,
        )(W_dist, b_dist.reshape(1, D3), emb_row, W_dens,
          b_dens.reshape(1, D3), W_out, b_out.reshape(1, EMBED_DIM))
        return pl.pallas_call(
            _block_kernel,
            grid=(Bv // BB,),
            in_specs=[
                pl.BlockSpec((3, N, BB), lambda i: (0, 0, i)),
                pl.BlockSpec((8, EMBED_DIM), lambda i: (0, 0)),
            ],
            out_specs=pl.BlockSpec((BB, N, EMBED_DIM), lambda i: (i, 0, 0)),
            out_shape=jax.ShapeDtypeStruct((Bv, N, EMBED_DIM), jnp.float32),
            interpret=interpret,
        )(pts_t, folded)
    return run


kernel = jax.jit(_build())


# speedup vs baseline: 1.0095x; 1.0095x over previous
import jax
import jax.numpy as jnp
from jax import lax
from jax.experimental import pallas as pl

EMBED_DIM = 384
D3 = EMBED_DIM // 3
N = 48
BB = 256

_INF = float("inf")


def _block_kernel(pts_ref, wdist_ref, bdist_ref, emb_ref,
                  wdens_ref, bdens_ref, wout_ref, bout_ref, out_ref):
    x = pts_ref[0]
    y = pts_ref[1]
    z = pts_ref[2]

    cx = jnp.mean(x, axis=0, keepdims=True)
    cy = jnp.mean(y, axis=0, keepdims=True)
    cz = jnp.mean(z, axis=0, keepdims=True)
    cd = jnp.sqrt((x - cx) ** 2 + (y - cy) ** 2 + (z - cz) ** 2)

    m1 = jnp.full((N, BB), _INF, dtype=jnp.float32)
    m2 = m1
    m3 = m1
    row = lax.broadcasted_iota(jnp.int32, (N, BB), 0)
    for j in range(N):
        dx = x - x[j:j + 1, :]
        dy = y - y[j:j + 1, :]
        dz = z - z[j:j + 1, :]
        dsq = dx * dx + dy * dy + dz * dz
        dsq = jnp.where(row == j, _INF, dsq)
        a = jnp.maximum(m1, dsq)
        m1 = jnp.minimum(m1, dsq)
        b = jnp.maximum(m2, dsq)
        m2 = jnp.minimum(m2, a)
        m3 = jnp.minimum(m3, b)
    ld = (jnp.sqrt(m1) + jnp.sqrt(m2) + jnp.sqrt(m3)) * (1.0 / 3.0)

    wout = wout_ref[...]
    w_lo = wout[0:D3, :]
    w_mid = wout[D3:2 * D3, :]
    w_hi = wout[2 * D3:3 * D3, :]
    v1 = jnp.dot(wdist_ref[...], w_lo, preferred_element_type=jnp.float32)
    v2 = jnp.dot(wdens_ref[...], w_hi, preferred_element_type=jnp.float32)
    cvec = (jnp.dot(bdist_ref[...], w_lo, preferred_element_type=jnp.float32)
            + jnp.dot(emb_ref[...], w_mid, preferred_element_type=jnp.float32)
            + jnp.dot(bdens_ref[...], w_hi, preferred_element_type=jnp.float32)
            + bout_ref[...])

    cd_t = jnp.transpose(cd, (1, 0))[:, :, None]
    ld_t = jnp.transpose(ld, (1, 0))[:, :, None]
    out_ref[...] = (cd_t * v1[None, :, :] + ld_t * v2[None, :, :]
                    + cvec[None, :, :])


def _build(interpret=False):
    def run(points, W_dist, b_dist, emb_count, W_dens, b_dens, W_out, b_out):
        Bv = points.shape[0]
        pts_t = jnp.transpose(points, (2, 1, 0))
        emb_row = emb_count[N:N + 1, :]
        return pl.pallas_call(
            _block_kernel,
            grid=(Bv // BB,),
            in_specs=[
                pl.BlockSpec((3, N, BB), lambda i: (0, 0, i)),
                pl.BlockSpec((1, D3), lambda i: (0, 0)),
                pl.BlockSpec((1, D3), lambda i: (0, 0)),
                pl.BlockSpec((1, D3), lambda i: (0, 0)),
                pl.BlockSpec((1, D3), lambda i: (0, 0)),
                pl.BlockSpec((1, D3), lambda i: (0, 0)),
                pl.BlockSpec((EMBED_DIM, EMBED_DIM), lambda i: (0, 0)),
                pl.BlockSpec((1, EMBED_DIM), lambda i: (0, 0)),
            ],
            out_specs=pl.BlockSpec((BB, N, EMBED_DIM), lambda i: (i, 0, 0)),
            out_shape=jax.ShapeDtypeStruct((Bv, N, EMBED_DIM), jnp.float32),
            interpret=interpret,
        )(pts_t, W_dist, b_dist.reshape(1, D3), emb_row,
          W_dens, b_dens.reshape(1, D3), W_out, b_out.reshape(1, EMBED_DIM))
    return run


kernel = jax.jit(_build())


# speedup vs baseline: 1.0114x; 1.0019x over previous
import jax
import jax.numpy as jnp
from jax import lax
from jax.experimental import pallas as pl

EMBED_DIM = 384
D3 = EMBED_DIM // 3
N = 48
BB = 256

_INF = float("inf")


def _block_kernel(pts_ref, wdist_ref, bdist_ref, emb_ref,
                  wdens_ref, bdens_ref, wout_ref, bout_ref, out_ref):
    x = pts_ref[0]
    y = pts_ref[1]
    z = pts_ref[2]

    cx = jnp.mean(x, axis=0, keepdims=True)
    cy = jnp.mean(y, axis=0, keepdims=True)
    cz = jnp.mean(z, axis=0, keepdims=True)
    cd = jnp.sqrt((x - cx) ** 2 + (y - cy) ** 2 + (z - cz) ** 2)

    m1 = jnp.full((N, BB), _INF, dtype=jnp.float32)
    m2 = m1
    m3 = m1
    row = lax.broadcasted_iota(jnp.int32, (N, BB), 0)
    for j in range(N):
        dx = x - x[j:j + 1, :]
        dy = y - y[j:j + 1, :]
        dz = z - z[j:j + 1, :]
        dsq = dx * dx + dy * dy + dz * dz
        dsq = jnp.where(row == j, _INF, dsq)
        a = jnp.maximum(m1, dsq)
        m1 = jnp.minimum(m1, dsq)
        b = jnp.maximum(m2, dsq)
        m2 = jnp.minimum(m2, a)
        m3 = jnp.minimum(m3, b)
    ld = (jnp.sqrt(m1) + jnp.sqrt(m2) + jnp.sqrt(m3)) * (1.0 / 3.0)

    wout = wout_ref[...]
    w_lo = wout[0:D3, :]
    w_mid = wout[D3:2 * D3, :]
    w_hi = wout[2 * D3:3 * D3, :]
    v1 = jnp.dot(wdist_ref[...], w_lo, preferred_element_type=jnp.float32)
    v2 = jnp.dot(wdens_ref[...], w_hi, preferred_element_type=jnp.float32)
    cvec = (jnp.dot(bdist_ref[...], w_lo, preferred_element_type=jnp.float32)
            + jnp.dot(emb_ref[...], w_mid, preferred_element_type=jnp.float32)
            + jnp.dot(bdens_ref[...], w_hi, preferred_element_type=jnp.float32)
            + bout_ref[...])

    cd_t = jnp.transpose(cd, (1, 0))[:, :, None]
    ld_t = jnp.transpose(ld, (1, 0))[:, :, None]
    out_ref[...] = (cd_t * v1[None, :, :] + ld_t * v2[None, :, :]
                    + cvec[None, :, :])


@jax.jit
def kernel(points, W_dist, b_dist, emb_count, W_dens, b_dens, W_out, b_out):
    Bv = points.shape[0]
    pts_t = jnp.transpose(points, (2, 1, 0))
    emb_row = emb_count[N:N + 1, :]
    return pl.pallas_call(
        _block_kernel,
        grid=(Bv // BB,),
        in_specs=[
            pl.BlockSpec((3, N, BB), lambda i: (0, 0, i)),
            pl.BlockSpec((1, D3), lambda i: (0, 0)),
            pl.BlockSpec((1, D3), lambda i: (0, 0)),
            pl.BlockSpec((1, D3), lambda i: (0, 0)),
            pl.BlockSpec((1, D3), lambda i: (0, 0)),
            pl.BlockSpec((1, D3), lambda i: (0, 0)),
            pl.BlockSpec((EMBED_DIM, EMBED_DIM), lambda i: (0, 0)),
            pl.BlockSpec((1, EMBED_DIM), lambda i: (0, 0)),
        ],
        out_specs=pl.BlockSpec((BB, N, EMBED_DIM), lambda i: (i, 0, 0)),
        out_shape=jax.ShapeDtypeStruct((Bv, N, EMBED_DIM), jnp.float32),
    )(pts_t, W_dist, b_dist.reshape(1, D3), emb_row,
      W_dens, b_dens.reshape(1, D3), W_out, b_out.reshape(1, EMBED_DIM))
